# baseline (speedup 1.0000x reference)
import jax
import jax.numpy as jnp
from jax import lax
from jax.experimental import pallas as pl
from jax.experimental.pallas import tpu as pltpu

B, N, NFEAT, NHID, NCLASS = 8, 2048, 128, 16, 128
NS = 2
RC = N // NS


def _gcn_body(*refs):
    (x_ref, *a_refs) = refs[:1 + NS]
    (w1_ref, b1_ref, w2_ref, b2_ref, wfc_ref, bfc_ref, out_ref) = refs[1 + NS:]
    xb = x_ref[0]
    s1 = jnp.dot(xb, w1_ref[...],
                 preferred_element_type=jnp.float32)
    s1b = s1.astype(jnp.bfloat16)
    hTs = [
        jnp.maximum(
            lax.dot_general(s1b, a_ref[0].astype(jnp.bfloat16),
                            (((0,), (1,)), ((), ())),
                            preferred_element_type=jnp.float32)
            + b1_ref[...], 0.0)
        for a_ref in a_refs
    ]
    s2_row = jnp.concatenate([
        lax.dot_general(w2_ref[...], hT, (((0,), (0,)), ((), ())),
                        preferred_element_type=jnp.float32)
        for hT in hTs], axis=1)
    s2b = s2_row.astype(jnp.bfloat16)
    logits = bfc_ref[...]
    for s, a_ref in enumerate(a_refs):
        g_part = lax.dot_general(s2b, a_ref[0].astype(jnp.bfloat16),
                                 (((1,), (1,)), ((), ())),
                                 preferred_element_type=jnp.float32) \
            + b2_ref[...]
        logits = logits + lax.dot_general(
            g_part, wfc_ref[:, s * RC:(s + 1) * RC],
            (((1,), (1,)), ((), ())),
            preferred_element_type=jnp.float32)
    m = jnp.max(logits, axis=1, keepdims=True)
    shifted = logits - m
    lse = jnp.log(jnp.sum(jnp.exp(shifted), axis=1, keepdims=True))
    out_ref[0] = shifted - lse


def kernel(x, adj, W1, b1, W2, b2, Wfc, bfc):
    adj_specs = [
        pl.BlockSpec((1, RC, N), lambda b, s=s: (b, s, 0)) for s in range(NS)
    ]
    out = pl.pallas_call(
        _gcn_body,
        grid=(B,),
        in_specs=[pl.BlockSpec((1, N, NFEAT), lambda b: (b, 0, 0))]
        + adj_specs
        + [
            pl.BlockSpec((NFEAT, NHID), lambda b: (0, 0)),
            pl.BlockSpec((NHID, 1), lambda b: (0, 0)),
            pl.BlockSpec((NHID, 1), lambda b: (0, 0)),
            pl.BlockSpec((1, 1), lambda b: (0, 0)),
            pl.BlockSpec((NCLASS, N), lambda b: (0, 0)),
            pl.BlockSpec((1, NCLASS), lambda b: (0, 0)),
        ],
        out_specs=pl.BlockSpec((1, 1, NCLASS), lambda b: (b, 0, 0)),
        out_shape=jax.ShapeDtypeStruct((B, 1, NCLASS), jnp.float32),
        compiler_params=pltpu.CompilerParams(
            dimension_semantics=("arbitrary",)),
    )(x, *([adj] * NS), W1, b1.reshape(NHID, 1), W2, b2.reshape(1, 1), Wfc,
      bfc.reshape(1, NCLASS))
    return out[:, 0, :]


# speedup vs baseline: 1.0480x; 1.0480x over previous
import jax
import jax.numpy as jnp
from jax import lax
from jax.experimental import pallas as pl
from jax.experimental.pallas import tpu as pltpu

B, N, NFEAT, NHID, NCLASS = 8, 2048, 128, 16, 128
NS = 1
RC = N // NS


def _gcn_body(*refs):
    (x_ref, *a_refs) = refs[:1 + NS]
    (w1_ref, b1_ref, w2_ref, b2_ref, wfc_ref, bfc_ref, out_ref) = refs[1 + NS:]
    xb = x_ref[0]
    s1 = jnp.dot(xb, w1_ref[...],
                 preferred_element_type=jnp.float32)
    s1b = s1.astype(jnp.bfloat16)
    hTs = [
        jnp.maximum(
            lax.dot_general(s1b, a_ref[0].astype(jnp.bfloat16),
                            (((0,), (1,)), ((), ())),
                            preferred_element_type=jnp.float32)
            + b1_ref[...], 0.0)
        for a_ref in a_refs
    ]
    s2_row = jnp.concatenate([
        lax.dot_general(w2_ref[...], hT, (((0,), (0,)), ((), ())),
                        preferred_element_type=jnp.float32)
        for hT in hTs], axis=1)
    s2b = s2_row.astype(jnp.bfloat16)
    logits = bfc_ref[...]
    for s, a_ref in enumerate(a_refs):
        g_part = lax.dot_general(s2b, a_ref[0].astype(jnp.bfloat16),
                                 (((1,), (1,)), ((), ())),
                                 preferred_element_type=jnp.float32) \
            + b2_ref[...]
        logits = logits + lax.dot_general(
            g_part, wfc_ref[:, s * RC:(s + 1) * RC],
            (((1,), (1,)), ((), ())),
            preferred_element_type=jnp.float32)
    m = jnp.max(logits, axis=1, keepdims=True)
    shifted = logits - m
    lse = jnp.log(jnp.sum(jnp.exp(shifted), axis=1, keepdims=True))
    out_ref[0] = shifted - lse


def kernel(x, adj, W1, b1, W2, b2, Wfc, bfc):
    adj_specs = [
        pl.BlockSpec((1, RC, N), lambda b, s=s: (b, s, 0)) for s in range(NS)
    ]
    out = pl.pallas_call(
        _gcn_body,
        grid=(B,),
        in_specs=[pl.BlockSpec((1, N, NFEAT), lambda b: (b, 0, 0))]
        + adj_specs
        + [
            pl.BlockSpec((NFEAT, NHID), lambda b: (0, 0)),
            pl.BlockSpec((NHID, 1), lambda b: (0, 0)),
            pl.BlockSpec((NHID, 1), lambda b: (0, 0)),
            pl.BlockSpec((1, 1), lambda b: (0, 0)),
            pl.BlockSpec((NCLASS, N), lambda b: (0, 0)),
            pl.BlockSpec((1, NCLASS), lambda b: (0, 0)),
        ],
        out_specs=pl.BlockSpec((1, 1, NCLASS), lambda b: (b, 0, 0)),
        out_shape=jax.ShapeDtypeStruct((B, 1, NCLASS), jnp.float32),
        compiler_params=pltpu.CompilerParams(
            dimension_semantics=("arbitrary",)),
    )(x, *([adj] * NS), W1, b1.reshape(NHID, 1), W2, b2.reshape(1, 1), Wfc,
      bfc.reshape(1, NCLASS))
    return out[:, 0, :]
